# baseline (speedup 1.0000x reference)
import functools

import jax
import jax.numpy as jnp
from jax import lax
from jax.experimental import pallas as pl
from jax.experimental.pallas import tpu as pltpu
from jax.experimental.pallas import tpu_sc as plsc

_NC = 2
_NS = 16
_NW = _NC * _NS
_TC = 32768


def _tc_reformat(uT, iT):
    D, N = uT.shape

    def body(u_ref, i_ref, o_ref):
        x2 = jnp.concatenate(
            [u_ref[...].astype(jnp.bfloat16),
             i_ref[...].astype(jnp.bfloat16)], axis=0)
        eye = (lax.broadcasted_iota(jnp.int32, (2 * D, 2 * D), 0)
               == lax.broadcasted_iota(jnp.int32, (2 * D, 2 * D), 1)
               ).astype(jnp.bfloat16)
        t = lax.dot_general(x2, eye, (((0,), (0,)), ((), ())),
                            preferred_element_type=jnp.float32)
        o_ref[...] = pltpu.bitcast(t.astype(jnp.bfloat16), jnp.int32)

    return pl.pallas_call(
        body,
        grid=(N + _TC - 1) // _TC,
        in_specs=[
            pl.BlockSpec((D, _TC), lambda g: (0, g)),
            pl.BlockSpec((D, _TC), lambda g: (0, g)),
        ],
        out_specs=pl.BlockSpec((_TC // 2, 2 * D), lambda g: (g, 0)),
        out_shape=jax.ShapeDtypeStruct((N // 2, 2 * D), jnp.int32),
        compiler_params=pltpu.CompilerParams(
            vmem_limit_bytes=100 * 1024 * 1024),
    )(uT, iT)


def _sc_gather(tui, ubp, ibp, uid, iid):
    B = uid.shape[0]
    W = tui.shape[1]
    bpw = B // _NW
    mesh = plsc.VectorSubcoreMesh(core_axis_name="c", subcore_axis_name="s")
    out_type = (
        jax.ShapeDtypeStruct((B, W), jnp.int32),
        jax.ShapeDtypeStruct((B, W), jnp.int32),
        jax.ShapeDtypeStruct((B, W), jnp.float32),
        jax.ShapeDtypeStruct((B, W), jnp.float32),
    )

    @functools.partial(
        pl.kernel,
        out_type=out_type,
        mesh=mesh,
        scratch_types=[
            pltpu.VMEM((bpw,), jnp.int32),
            pltpu.VMEM((bpw,), jnp.int32),
            pltpu.VMEM((bpw,), jnp.int32),
            pltpu.VMEM((bpw,), jnp.int32),
            pltpu.VMEM((bpw,), jnp.int32),
            pltpu.VMEM((bpw,), jnp.int32),
            pltpu.VMEM((bpw, W), jnp.int32),
            pltpu.VMEM((bpw, W), jnp.int32),
            pltpu.VMEM((bpw, W), jnp.float32),
            pltpu.VMEM((bpw, W), jnp.float32),
            pltpu.SemaphoreType.DMA,
            pltpu.SemaphoreType.DMA,
            pltpu.SemaphoreType.DMA,
            pltpu.SemaphoreType.DMA,
        ],
    )
    def k(tui_h, ubp_h, ibp_h, uid_h, iid_h,
          ou_h, oi_h, oub_h, oib_h,
          uidx_v, iidx_v, uer_v, ier_v, urow_v, irow_v, uw_v, iw_v,
          ubw_v, ibw_v, s0, s1, s2, s3):
        wid = lax.axis_index("s") * _NC + lax.axis_index("c")
        base = wid * bpw
        pltpu.sync_copy(uid_h.at[pl.ds(base, bpw)], uidx_v)
        pltpu.sync_copy(iid_h.at[pl.ds(base, bpw)], iidx_v)

        @pl.loop(0, bpw, step=16)
        def _(j):
            uer_v[pl.ds(j, 16)] = jnp.right_shift(uidx_v[pl.ds(j, 16)], 1)
            ier_v[pl.ds(j, 16)] = jnp.right_shift(iidx_v[pl.ds(j, 16)], 1)
            urow_v[pl.ds(j, 16)] = jnp.right_shift(uidx_v[pl.ds(j, 16)], 7)
            irow_v[pl.ds(j, 16)] = jnp.right_shift(iidx_v[pl.ds(j, 16)], 7)

        c0 = pltpu.async_copy(tui_h.at[uer_v], uw_v, s0)
        c1 = pltpu.async_copy(tui_h.at[ier_v], iw_v, s1)
        c2 = pltpu.async_copy(ubp_h.at[urow_v], ubw_v, s2)
        c3 = pltpu.async_copy(ibp_h.at[irow_v], ibw_v, s3)

        c0.wait()
        c1.wait()
        c2.wait()
        c3.wait()
        pltpu.sync_copy(uw_v, ou_h.at[pl.ds(base, bpw)])
        pltpu.sync_copy(iw_v, oi_h.at[pl.ds(base, bpw)])
        pltpu.sync_copy(ubw_v, oub_h.at[pl.ds(base, bpw)])
        pltpu.sync_copy(ibw_v, oib_h.at[pl.ds(base, bpw)])

    return k(tui, ubp, ibp, uid, iid)


def _unpack(w, par):
    bits = jnp.where(par == 0, w & 0xFFFF, lax.shift_right_logical(w, 16))
    return lax.bitcast_convert_type(lax.shift_left(bits, 16), jnp.float32)


def _tc_select(uw, iw, ubw, ibw, uid2, iid2):
    B, W = ubw.shape
    D = W // 2
    TM = 2048

    def body(uw_ref, iw_ref, ubw_ref, ibw_ref, uid_ref, iid_ref,
             ou_ref, oi_ref, oub_ref, oib_ref):
        up = uid_ref[...] & 1
        ip = iid_ref[...] & 1
        ou_ref[...] = _unpack(uw_ref[...], up)[:, :D].astype(jnp.bfloat16)
        oi_ref[...] = _unpack(iw_ref[...], ip)[:, D:].astype(jnp.bfloat16)
        lane = lax.broadcasted_iota(jnp.int32, (TM, W), 1)
        oub_ref[...] = jnp.sum(
            jnp.where(lane == (uid_ref[...] & (W - 1)), ubw_ref[...], 0.0),
            axis=1, keepdims=True)
        oib_ref[...] = jnp.sum(
            jnp.where(lane == (iid_ref[...] & (W - 1)), ibw_ref[...], 0.0),
            axis=1, keepdims=True)

    return pl.pallas_call(
        body,
        grid=(B // TM,),
        in_specs=[
            pl.BlockSpec((TM, W), lambda m: (m, 0)),
            pl.BlockSpec((TM, W), lambda m: (m, 0)),
            pl.BlockSpec((TM, W), lambda m: (m, 0)),
            pl.BlockSpec((TM, W), lambda m: (m, 0)),
            pl.BlockSpec((TM, 1), lambda m: (m, 0)),
            pl.BlockSpec((TM, 1), lambda m: (m, 0)),
        ],
        out_specs=[
            pl.BlockSpec((TM, D), lambda m: (m, 0)),
            pl.BlockSpec((TM, D), lambda m: (m, 0)),
            pl.BlockSpec((TM, 1), lambda m: (m, 0)),
            pl.BlockSpec((TM, 1), lambda m: (m, 0)),
        ],
        out_shape=[
            jax.ShapeDtypeStruct((B, D), jnp.bfloat16),
            jax.ShapeDtypeStruct((B, D), jnp.bfloat16),
            jax.ShapeDtypeStruct((B, 1), jnp.float32),
            jax.ShapeDtypeStruct((B, 1), jnp.float32),
        ],
    )(uw, iw, ubw, ibw, uid2, iid2)


def _tc_matmul(u, it, ubg, ibg_row):
    B, D = u.shape
    TM = 1024

    def body(u_ref, i_ref, ub_ref, ib_ref, o_ref):
        acc = lax.dot_general(
            u_ref[...], i_ref[...],
            (((1,), (1,)), ((), ())),
            preferred_element_type=jnp.float32,
        )
        o_ref[...] = acc + ub_ref[...] + ib_ref[...]

    return pl.pallas_call(
        body,
        grid=(B // TM,),
        in_specs=[
            pl.BlockSpec((TM, D), lambda m: (m, 0)),
            pl.BlockSpec((B, D), lambda m: (0, 0)),
            pl.BlockSpec((TM, 1), lambda m: (m, 0)),
            pl.BlockSpec((1, B), lambda m: (0, 0)),
        ],
        out_specs=pl.BlockSpec((TM, B), lambda m: (m, 0)),
        out_shape=jax.ShapeDtypeStruct((B, B), jnp.float32),
    )(u, it, ubg, ibg_row)


def kernel(user_id, item_id, user_emb, item_emb, ub, ib):
    uid = user_id.astype(jnp.int32)
    iid = item_id.astype(jnp.int32)
    n = ub.shape[0]
    npad = (-n) % 128
    ubp = jnp.concatenate(
        [ub.reshape(1, n), jnp.zeros((1, npad), jnp.float32)], axis=1
    ).reshape(-1, 128)
    ibp = jnp.concatenate(
        [ib.reshape(1, n), jnp.zeros((1, npad), jnp.float32)], axis=1
    ).reshape(-1, 128)
    tui = _tc_reformat(user_emb.T, item_emb.T)
    uw, iw, ubw, ibw = _sc_gather(tui, ubp, ibp, uid, iid)
    uid2 = uid.reshape(-1, 1)
    iid2 = iid.reshape(-1, 1)
    usel, isel, ubsel, ibsel = _tc_select(uw, iw, ubw, ibw, uid2, iid2)
    return _tc_matmul(usel, isel, ubsel, ibsel.reshape(1, -1))


# speedup vs baseline: 1.0040x; 1.0040x over previous
import functools

import jax
import jax.numpy as jnp
from jax import lax
from jax.experimental import pallas as pl
from jax.experimental.pallas import tpu as pltpu
from jax.experimental.pallas import tpu_sc as plsc

_NC = 2
_NS = 16
_NW = _NC * _NS
_TC = 32768


def _tc_reformat(uT, iT):
    D, N = uT.shape

    def body(u_ref, i_ref, o_ref):
        x2 = jnp.concatenate(
            [u_ref[...].astype(jnp.bfloat16),
             i_ref[...].astype(jnp.bfloat16)], axis=0)
        eye = (lax.broadcasted_iota(jnp.int32, (2 * D, 2 * D), 0)
               == lax.broadcasted_iota(jnp.int32, (2 * D, 2 * D), 1)
               ).astype(jnp.bfloat16)
        t = lax.dot_general(x2, eye, (((0,), (0,)), ((), ())),
                            preferred_element_type=jnp.float32)
        o_ref[...] = pltpu.bitcast(t.astype(jnp.bfloat16), jnp.int32)

    return pl.pallas_call(
        body,
        grid=(N + _TC - 1) // _TC,
        in_specs=[
            pl.BlockSpec((D, _TC), lambda g: (0, g)),
            pl.BlockSpec((D, _TC), lambda g: (0, g)),
        ],
        out_specs=pl.BlockSpec((_TC // 2, 2 * D), lambda g: (g, 0)),
        out_shape=jax.ShapeDtypeStruct((N // 2, 2 * D), jnp.int32),
        compiler_params=pltpu.CompilerParams(
            vmem_limit_bytes=100 * 1024 * 1024),
    )(uT, iT)


def _sc_gather(tui, ubp, ibp, uid, iid):
    B = uid.shape[0]
    W = tui.shape[1]
    bpw = B // _NW
    mesh = plsc.VectorSubcoreMesh(core_axis_name="c", subcore_axis_name="s")
    out_type = (
        jax.ShapeDtypeStruct((B, W), jnp.int32),
        jax.ShapeDtypeStruct((B, W), jnp.int32),
        jax.ShapeDtypeStruct((B, W), jnp.float32),
        jax.ShapeDtypeStruct((B, W), jnp.float32),
    )

    @functools.partial(
        pl.kernel,
        out_type=out_type,
        mesh=mesh,
        scratch_types=[
            pltpu.VMEM((bpw,), jnp.int32),
            pltpu.VMEM((bpw,), jnp.int32),
            pltpu.VMEM((bpw,), jnp.int32),
            pltpu.VMEM((bpw,), jnp.int32),
            pltpu.VMEM((bpw,), jnp.int32),
            pltpu.VMEM((bpw,), jnp.int32),
            pltpu.VMEM((bpw, W), jnp.int32),
            pltpu.VMEM((bpw, W), jnp.int32),
            pltpu.VMEM((bpw, W), jnp.float32),
            pltpu.VMEM((bpw, W), jnp.float32),
            pltpu.SemaphoreType.DMA,
            pltpu.SemaphoreType.DMA,
            pltpu.SemaphoreType.DMA,
            pltpu.SemaphoreType.DMA,
        ],
    )
    def k(tui_h, ubp_h, ibp_h, uid_h, iid_h,
          ou_h, oi_h, oub_h, oib_h,
          uidx_v, iidx_v, uer_v, ier_v, urow_v, irow_v, uw_v, iw_v,
          ubw_v, ibw_v, s0, s1, s2, s3):
        wid = lax.axis_index("s") * _NC + lax.axis_index("c")
        base = wid * bpw
        pltpu.sync_copy(uid_h.at[pl.ds(base, bpw)], uidx_v)
        pltpu.sync_copy(iid_h.at[pl.ds(base, bpw)], iidx_v)

        @pl.loop(0, bpw, step=16)
        def _(j):
            uer_v[pl.ds(j, 16)] = jnp.right_shift(uidx_v[pl.ds(j, 16)], 1)
            ier_v[pl.ds(j, 16)] = jnp.right_shift(iidx_v[pl.ds(j, 16)], 1)
            urow_v[pl.ds(j, 16)] = jnp.right_shift(uidx_v[pl.ds(j, 16)], 7)
            irow_v[pl.ds(j, 16)] = jnp.right_shift(iidx_v[pl.ds(j, 16)], 7)

        c0 = pltpu.async_copy(tui_h.at[uer_v], uw_v, s0)
        c1 = pltpu.async_copy(tui_h.at[ier_v], iw_v, s1)
        c2 = pltpu.async_copy(ubp_h.at[urow_v], ubw_v, s2)
        c3 = pltpu.async_copy(ibp_h.at[irow_v], ibw_v, s3)

        c0.wait()
        c1.wait()
        c2.wait()
        c3.wait()
        pltpu.sync_copy(uw_v, ou_h.at[pl.ds(base, bpw)])
        pltpu.sync_copy(iw_v, oi_h.at[pl.ds(base, bpw)])
        pltpu.sync_copy(ubw_v, oub_h.at[pl.ds(base, bpw)])
        pltpu.sync_copy(ibw_v, oib_h.at[pl.ds(base, bpw)])

    return k(tui, ubp, ibp, uid, iid)


def _unpack(w, par):
    bits = jnp.where(par == 0, w & 0xFFFF, lax.shift_right_logical(w, 16))
    return lax.bitcast_convert_type(lax.shift_left(bits, 16), jnp.float32)


def _tc_select(uw, iw, ubw, ibw, uid2, iid2):
    B, W = ubw.shape
    D = W // 2
    TM = 2048

    def body(uw_ref, iw_ref, ubw_ref, ibw_ref, uid_ref, iid_ref,
             ou_ref, oi_ref, oub_ref, oib_ref):
        up = uid_ref[...] & 1
        ip = iid_ref[...] & 1
        ou_ref[...] = _unpack(uw_ref[...], up)[:, :D].astype(jnp.bfloat16)
        oi_ref[...] = _unpack(iw_ref[...], ip)[:, D:].astype(jnp.bfloat16)
        lane = lax.broadcasted_iota(jnp.int32, (TM, W), 1)
        oub_ref[...] = jnp.sum(
            jnp.where(lane == (uid_ref[...] & (W - 1)), ubw_ref[...], 0.0),
            axis=1, keepdims=True)
        oib_ref[...] = jnp.sum(
            jnp.where(lane == (iid_ref[...] & (W - 1)), ibw_ref[...], 0.0),
            axis=1, keepdims=True)

    return pl.pallas_call(
        body,
        grid=(B // TM,),
        in_specs=[
            pl.BlockSpec((TM, W), lambda m: (m, 0)),
            pl.BlockSpec((TM, W), lambda m: (m, 0)),
            pl.BlockSpec((TM, W), lambda m: (m, 0)),
            pl.BlockSpec((TM, W), lambda m: (m, 0)),
            pl.BlockSpec((TM, 1), lambda m: (m, 0)),
            pl.BlockSpec((TM, 1), lambda m: (m, 0)),
        ],
        out_specs=[
            pl.BlockSpec((TM, D), lambda m: (m, 0)),
            pl.BlockSpec((TM, D), lambda m: (m, 0)),
            pl.BlockSpec((TM, 1), lambda m: (m, 0)),
            pl.BlockSpec((TM, 1), lambda m: (m, 0)),
        ],
        out_shape=[
            jax.ShapeDtypeStruct((B, D), jnp.bfloat16),
            jax.ShapeDtypeStruct((B, D), jnp.bfloat16),
            jax.ShapeDtypeStruct((B, 1), jnp.float32),
            jax.ShapeDtypeStruct((B, 1), jnp.float32),
        ],
    )(uw, iw, ubw, ibw, uid2, iid2)


def _tc_matmul(u, it, ubg, ibg_row):
    B, D = u.shape
    TM = 512

    def body(u_ref, i_ref, ub_ref, ib_ref, o_ref):
        acc = lax.dot_general(
            u_ref[...], i_ref[...],
            (((1,), (1,)), ((), ())),
            preferred_element_type=jnp.float32,
        )
        o_ref[...] = acc + ub_ref[...] + ib_ref[...]

    return pl.pallas_call(
        body,
        grid=(B // TM,),
        in_specs=[
            pl.BlockSpec((TM, D), lambda m: (m, 0)),
            pl.BlockSpec((B, D), lambda m: (0, 0)),
            pl.BlockSpec((TM, 1), lambda m: (m, 0)),
            pl.BlockSpec((1, B), lambda m: (0, 0)),
        ],
        out_specs=pl.BlockSpec((TM, B), lambda m: (m, 0)),
        out_shape=jax.ShapeDtypeStruct((B, B), jnp.float32),
    )(u, it, ubg, ibg_row)


def kernel(user_id, item_id, user_emb, item_emb, ub, ib):
    uid = user_id.astype(jnp.int32)
    iid = item_id.astype(jnp.int32)
    n = ub.shape[0]
    npad = (-n) % 128
    ubp = jnp.concatenate(
        [ub.reshape(1, n), jnp.zeros((1, npad), jnp.float32)], axis=1
    ).reshape(-1, 128)
    ibp = jnp.concatenate(
        [ib.reshape(1, n), jnp.zeros((1, npad), jnp.float32)], axis=1
    ).reshape(-1, 128)
    tui = _tc_reformat(user_emb.T, item_emb.T)
    uw, iw, ubw, ibw = _sc_gather(tui, ubp, ibp, uid, iid)
    uid2 = uid.reshape(-1, 1)
    iid2 = iid.reshape(-1, 1)
    usel, isel, ubsel, ibsel = _tc_select(uw, iw, ubw, ibw, uid2, iid2)
    return _tc_matmul(usel, isel, ubsel, ibsel.reshape(1, -1))
